# baseline (speedup 1.0000x reference)
import functools

import jax
import jax.numpy as jnp
from jax.experimental import pallas as pl


def _support_body(x_ref, wl_ref, wh_ref, sl_ref, sh_ref):
    xv = x_ref[...]
    sl_ref[...] = jnp.dot(
        xv, wl_ref[...], preferred_element_type=jnp.float32,
        precision=jax.lax.Precision.HIGHEST).astype(jnp.bfloat16)
    sh_ref[...] = jnp.dot(
        xv, wh_ref[...], preferred_element_type=jnp.float32,
        precision=jax.lax.Precision.HIGHEST).astype(jnp.bfloat16)


def _support(x, wl, wh):
    n, _ = x.shape
    h = wl.shape[1]
    return pl.pallas_call(
        _support_body,
        out_shape=(
            jax.ShapeDtypeStruct((n, h), jnp.bfloat16),
            jax.ShapeDtypeStruct((n, h), jnp.bfloat16),
        ),
    )(x, wl, wh)


def _aggregate_body(adj_ref, adjh_ref, sl_ref, sh_ref, b_ref, out_ref, *, relu):
    a = adj_ref[...].astype(jnp.bfloat16)
    ah = adjh_ref[...].astype(jnp.bfloat16)
    acc = jnp.dot(a, sl_ref[...], preferred_element_type=jnp.float32)
    acc = acc + jnp.dot(ah, sh_ref[...], preferred_element_type=jnp.float32)
    acc = acc + b_ref[...]
    if relu:
        acc = jnp.maximum(acc, 0.0)
    out_ref[...] = acc


def _aggregate(adj, adj_high, s_low, s_high, b, relu, block_rows=200):
    n = adj.shape[0]
    h = s_low.shape[1]
    grid = (n // block_rows,)
    return pl.pallas_call(
        functools.partial(_aggregate_body, relu=relu),
        grid=grid,
        in_specs=[
            pl.BlockSpec((block_rows, n), lambda i: (i, 0)),
            pl.BlockSpec((block_rows, n), lambda i: (i, 0)),
            pl.BlockSpec((n, h), lambda i: (0, 0)),
            pl.BlockSpec((n, h), lambda i: (0, 0)),
            pl.BlockSpec((1, h), lambda i: (0, 0)),
        ],
        out_specs=pl.BlockSpec((block_rows, h), lambda i: (i, 0)),
        out_shape=jax.ShapeDtypeStruct((n, h), jnp.float32),
    )(adj, adj_high, s_low, s_high, b)


def kernel(x, adj, adj_high, W0_low, W0_high, b0, W1_low, W1_high, b1):
    s0l, s0h = _support(x, W0_low, W0_high)
    fea = _aggregate(adj, adj_high, s0l, s0h, b0.reshape(1, -1), relu=True)
    s1l, s1h = _support(fea, W1_low, W1_high)
    out = _aggregate(adj, adj_high, s1l, s1h, b1.reshape(1, -1), relu=False)
    return out
